# baseline (speedup 1.0000x reference)
import functools

import jax
import jax.numpy as jnp
from jax import lax
from jax.experimental import pallas as pl
from jax.experimental.pallas import tpu as pltpu
from jax.experimental.pallas import tpu_sc as plsc

NC, NS, LANES = 2, 16, 16
NW = NC * NS


def _sc_gather_sum(B, L, V, D):
    CB = 64
    n_chunks = B // CB
    n_groups = CB // LANES
    d_per = D // NW
    CHUNK = CB * L

    mesh = plsc.VectorSubcoreMesh(
        core_axis_name="c", subcore_axis_name="s", num_cores=NC, num_subcores=NS
    )

    @functools.partial(
        pl.kernel,
        out_type=jax.ShapeDtypeStruct((D, B), jnp.float32),
        mesh=mesh,
        compiler_params=pltpu.CompilerParams(
            use_tc_tiling_on_sc=False, needs_layout_passes=False
        ),
        scratch_types=[
            pltpu.VMEM((V,), jnp.float32),
            pltpu.VMEM((2, CHUNK), jnp.int32),
            pltpu.VMEM((B,), jnp.float32),
            pltpu.VMEM((LANES,), jnp.float32),
            pltpu.SemaphoreType.DMA,
            pltpu.SemaphoreType.DMA,
        ],
    )
    def k(tok_hbm, w_hbm, bb_hbm, out_hbm, wrow_v, tok_v, orow_v, bias_v,
          sem0, sem1):
        cid = lax.axis_index("c")
        sid = lax.axis_index("s")
        wid = sid * NC + cid
        sems = (sem0, sem1)
        base = lax.iota(jnp.int32, 16) * L
        boffs = [base + g * LANES * L for g in range(n_groups)]

        for p in range(d_per):
            d = wid * d_per + p
            pltpu.sync_copy(w_hbm.at[d], wrow_v)
            pltpu.sync_copy(bb_hbm.at[d], bias_v)
            bias = bias_v[...]
            pending = pltpu.async_copy(
                tok_hbm.at[pl.ds(0, CHUNK)], tok_v.at[0], sems[0])
            for c in range(n_chunks):
                buf = c % 2
                nxt = None
                if c + 1 < n_chunks:
                    nxt = pltpu.async_copy(
                        tok_hbm.at[pl.ds((c + 1) * CHUNK, CHUNK)],
                        tok_v.at[(c + 1) % 2], sems[(c + 1) % 2])
                pending.wait()

                def lbody(l, accs, _buf=buf):
                    new = []
                    for g in range(n_groups):
                        ti = plsc.load_gather(tok_v.at[_buf], [boffs[g] + l])
                        new.append(accs[g] + plsc.load_gather(wrow_v, [ti]))
                    return tuple(new)
                accs = lax.fori_loop(0, L, lbody, (bias,) * n_groups,
                                     unroll=4)
                for g in range(n_groups):
                    orow_v[pl.ds(c * CB + g * LANES, LANES)] = accs[g]
                pending = nxt
            pltpu.sync_copy(orow_v, out_hbm.at[d])

    return k


def kernel(tokens, W, b):
    B, L = tokens.shape
    D, V = W.shape
    tok_flat = tokens.astype(jnp.int32).reshape(B * L)
    bb = jnp.broadcast_to(b[:, None], (D, LANES))
    outT = _sc_gather_sum(B, L, V, D)(tok_flat, W, bb)
    return outT.T[:, None, :]


# speedup vs baseline: 1.1966x; 1.1966x over previous
import functools

import jax
import jax.numpy as jnp
from jax import lax
from jax.experimental import pallas as pl
from jax.experimental.pallas import tpu as pltpu
from jax.experimental.pallas import tpu_sc as plsc

NC, NS, LANES = 2, 16, 16
NW = NC * NS


def _sc_gather_sum(B, L, V, D):
    CB = 64
    n_chunks = B // CB
    n_groups = CB // LANES
    d_per = D // NW
    LP = L + 1
    CHUNK = CB * LP

    mesh = plsc.VectorSubcoreMesh(
        core_axis_name="c", subcore_axis_name="s", num_cores=NC, num_subcores=NS
    )

    @functools.partial(
        pl.kernel,
        out_type=jax.ShapeDtypeStruct((D, B), jnp.float32),
        mesh=mesh,
        compiler_params=pltpu.CompilerParams(
            use_tc_tiling_on_sc=False, needs_layout_passes=False,
            disable_bounds_checks=True
        ),
        scratch_types=[
            pltpu.VMEM((V,), jnp.float32),
            pltpu.VMEM((2, CHUNK), jnp.int32),
            pltpu.VMEM((B,), jnp.float32),
            pltpu.VMEM((LANES,), jnp.float32),
            pltpu.SemaphoreType.DMA,
            pltpu.SemaphoreType.DMA,
        ],
    )
    def k(tok_hbm, w_hbm, bb_hbm, out_hbm, wrow_v, tok_v, orow_v, bias_v,
          sem0, sem1):
        cid = lax.axis_index("c")
        sid = lax.axis_index("s")
        wid = sid * NC + cid
        sems = (sem0, sem1)
        base = lax.iota(jnp.int32, 16) * LP
        boffs = [base + g * LANES * LP for g in range(n_groups)]

        for p in range(d_per):
            d = wid * d_per + p
            pltpu.sync_copy(w_hbm.at[d], wrow_v)
            pltpu.sync_copy(bb_hbm.at[d], bias_v)
            bias = bias_v[...]
            pending = pltpu.async_copy(
                tok_hbm.at[pl.ds(0, CHUNK)], tok_v.at[0], sems[0])
            for c in range(n_chunks):
                buf = c % 2
                nxt = None
                if c + 1 < n_chunks:
                    nxt = pltpu.async_copy(
                        tok_hbm.at[pl.ds((c + 1) * CHUNK, CHUNK)],
                        tok_v.at[(c + 1) % 2], sems[(c + 1) % 2])
                pending.wait()

                def lbody(l, accs, _buf=buf):
                    new = []
                    for g in range(n_groups):
                        ti = plsc.load_gather(tok_v.at[_buf], [boffs[g] + l])
                        new.append(accs[g] + plsc.load_gather(wrow_v, [ti]))
                    return tuple(new)
                accs = lax.fori_loop(0, L, lbody, (bias,) * n_groups,
                                     unroll=2)
                for g in range(n_groups):
                    orow_v[pl.ds(c * CB + g * LANES, LANES)] = accs[g]
                pending = nxt
            pltpu.sync_copy(orow_v, out_hbm.at[d])

    return k


def kernel(tokens, W, b):
    B, L = tokens.shape
    D, V = W.shape
    tok_pad = jnp.pad(tokens.astype(jnp.int32), ((0, 0), (0, 1)))
    tok_flat = tok_pad.reshape(B * (L + 1))
    bb = jnp.broadcast_to(b[:, None], (D, LANES))
    outT = _sc_gather_sum(B, L, V, D)(tok_flat, W, bb)
    return outT.T[:, None, :]
